# baseline (speedup 1.0000x reference)
import functools

import jax
import jax.numpy as jnp
from jax import lax
from jax.experimental import pallas as pl
from jax.experimental.pallas import tpu as pltpu
from jax.experimental.pallas import tpu_sc as plsc

N = 10000
E = 320000
NC = 2
NS = 16
NW = NC * NS
C = 125
NCH = E // C
CH_E = NCH // NW
G = CH_E // 2
SZ = 632
NP = NS * SZ
BLK = 1000
GRID = N // BLK

_MESH = dict(core_axis_name="c", subcore_axis_name="s")


def _deg_body(col_hbm, zeros_hbm, ones_hbm, out_hbm, colbuf, ones_v, acc, sem0, sem1):
    c = lax.axis_index("c")
    s = lax.axis_index("s")
    w = s * NC + c
    pltpu.async_copy(zeros_hbm, acc.at[pl.ds(s * SZ, SZ)], sem0)
    pltpu.sync_copy(col_hbm.at[pl.ds(w * CH_E, CH_E)], colbuf)
    pltpu.sync_copy(ones_hbm, ones_v)
    pltpu.make_async_copy(zeros_hbm, acc.at[pl.ds(s * SZ, SZ)], sem0).wait()
    plsc.subcore_barrier()

    pltpu.async_copy(ones_v, acc.at[colbuf.at[0]], sem0, add=True)

    def body(k, carry):
        i = 2 * k
        pltpu.async_copy(ones_v, acc.at[colbuf.at[i + 1]], sem1, add=True)
        pltpu.make_async_copy(ones_v, acc.at[colbuf.at[i]], sem0).wait()

        @pl.when(k + 1 < CH_E // 2)
        def _():
            pltpu.async_copy(ones_v, acc.at[colbuf.at[i + 2]], sem0, add=True)

        pltpu.make_async_copy(ones_v, acc.at[colbuf.at[i + 1]], sem1).wait()
        return carry

    lax.fori_loop(0, CH_E // 2, body, 0)
    plsc.subcore_barrier()
    pltpu.sync_copy(acc.at[pl.ds(s * SZ, SZ)], out_hbm.at[c, pl.ds(s * SZ, SZ)])


def _make_deg():
    return pl.kernel(
        _deg_body,
        out_type=jax.ShapeDtypeStruct((NC, NP, 16), jnp.float32),
        mesh=plsc.VectorSubcoreMesh(**_MESH),
        scratch_types=[
            pltpu.VMEM((CH_E, C), jnp.int32),
            pltpu.VMEM((C, 16), jnp.float32),
            pltpu.VMEM_SHARED((NP, 16), jnp.float32),
            pltpu.SemaphoreType.DMA,
            pltpu.SemaphoreType.DMA,
        ],
    )


def _agg_loop(table, rowbuf, colbuf, rows0, rows1, acc, sem0, sem1, nchunks):
    pltpu.async_copy(table.at[rowbuf.at[0]], rows0, sem0)

    def body(k, carry):
        i = 2 * k
        pltpu.async_copy(table.at[rowbuf.at[i + 1]], rows1, sem1)
        pltpu.make_async_copy(table.at[rowbuf.at[i]], rows0, sem0).wait()
        pltpu.sync_copy(rows0, acc.at[colbuf.at[i]], add=True)

        @pl.when(k + 1 < nchunks // 2)
        def _():
            pltpu.async_copy(table.at[rowbuf.at[i + 2]], rows0, sem0)

        pltpu.make_async_copy(table.at[rowbuf.at[i + 1]], rows1, sem1).wait()
        pltpu.sync_copy(rows1, acc.at[colbuf.at[i + 1]], add=True)
        return carry

    lax.fori_loop(0, nchunks // 2, body, 0)


def _agg128_body(hp_hbm, row_hbm, col_hbm, zeros_hbm, out_hbm,
                 rowbuf, colbuf, rows0, rows1, acc, sem0, sem1):
    c = lax.axis_index("c")
    s = lax.axis_index("s")
    w = s * NC + c
    pltpu.async_copy(zeros_hbm, acc.at[pl.ds(s * SZ, SZ)], sem0)
    pltpu.sync_copy(row_hbm.at[pl.ds(w * CH_E, G)], rowbuf)
    pltpu.sync_copy(col_hbm.at[pl.ds(w * CH_E, G)], colbuf)
    pltpu.make_async_copy(zeros_hbm, acc.at[pl.ds(s * SZ, SZ)], sem0).wait()
    plsc.subcore_barrier()
    for g in range(2):
        if g:
            base = w * CH_E + g * G
            pltpu.sync_copy(row_hbm.at[pl.ds(base, G)], rowbuf)
            pltpu.sync_copy(col_hbm.at[pl.ds(base, G)], colbuf)
        _agg_loop(hp_hbm, rowbuf, colbuf, rows0, rows1, acc, sem0, sem1, G)
    plsc.subcore_barrier()
    pltpu.sync_copy(acc.at[pl.ds(s * SZ, SZ)], out_hbm.at[c, pl.ds(s * SZ, SZ)])


def _make_agg128():
    return pl.kernel(
        _agg128_body,
        out_type=jax.ShapeDtypeStruct((NC, NP, 128), jnp.float32),
        mesh=plsc.VectorSubcoreMesh(**_MESH),
        scratch_types=[
            pltpu.VMEM((G, C), jnp.int32),
            pltpu.VMEM((G, C), jnp.int32),
            pltpu.VMEM((C, 128), jnp.float32),
            pltpu.VMEM((C, 128), jnp.float32),
            pltpu.VMEM_SHARED((NP, 128), jnp.float32),
            pltpu.SemaphoreType.DMA,
            pltpu.SemaphoreType.DMA,
        ],
    )


def _agg64_body(hp_hbm, row_hbm, col_hbm, zeros_hbm, out_hbm,
                rowbuf, colbuf, rows0, rows1, acc, sem0, sem1):
    c = lax.axis_index("c")
    s = lax.axis_index("s")
    w = s * NC + c
    pltpu.async_copy(zeros_hbm, acc.at[pl.ds(s * SZ, SZ)], sem0)
    pltpu.sync_copy(row_hbm.at[pl.ds(w * CH_E, CH_E)], rowbuf)
    pltpu.sync_copy(col_hbm.at[pl.ds(w * CH_E, CH_E)], colbuf)
    pltpu.make_async_copy(zeros_hbm, acc.at[pl.ds(s * SZ, SZ)], sem0).wait()
    plsc.subcore_barrier()
    _agg_loop(hp_hbm, rowbuf, colbuf, rows0, rows1, acc, sem0, sem1, CH_E)
    plsc.subcore_barrier()
    pltpu.sync_copy(acc.at[pl.ds(s * SZ, SZ)], out_hbm.at[c, pl.ds(s * SZ, SZ)])


def _make_agg64():
    return pl.kernel(
        _agg64_body,
        out_type=jax.ShapeDtypeStruct((NC, NP, 64), jnp.float32),
        mesh=plsc.VectorSubcoreMesh(**_MESH),
        compiler_params=pltpu.CompilerParams(use_tc_tiling_on_sc=False),
        scratch_types=[
            pltpu.VMEM((CH_E, C), jnp.int32),
            pltpu.VMEM((CH_E, C), jnp.int32),
            pltpu.VMEM((C, 64), jnp.float32),
            pltpu.VMEM((C, 64), jnp.float32),
            pltpu.VMEM_SHARED((NP, 64), jnp.float32),
            pltpu.SemaphoreType.DMA,
            pltpu.SemaphoreType.DMA,
        ],
    )


def _tc1_body(x_ref, w_ref, degp_ref, hp_ref, dis_ref):
    d = degp_ref[...]
    deg = 1.0 + d[0] + d[1]
    dis16 = lax.rsqrt(deg)
    h = jnp.dot(x_ref[...], w_ref[...], preferred_element_type=jnp.float32)
    hp_ref[...] = h * dis16[:, 0:1]
    dis_ref[...] = dis16


def _tc_mid_body(p_ref, hp_ref, dis_ref, b_ref, w_ref, o_ref):
    p = p_ref[...]
    dis = dis_ref[...][:, 0:1]
    z = (p[0] + p[1] + hp_ref[...]) * dis + b_ref[...]
    a = jnp.maximum(z, 0.0)
    o_ref[...] = jnp.dot(a, w_ref[...], preferred_element_type=jnp.float32) * dis


def _tc_final_body(p_ref, hp_ref, dis_ref, b_ref, o_ref):
    p = p_ref[...]
    dis = dis_ref[...][:, 0:1]
    o_ref[...] = (p[0] + p[1] + hp_ref[...]) * dis + b_ref[...]


def _tc1(x, w, degp):
    return pl.pallas_call(
        _tc1_body,
        grid=(GRID,),
        in_specs=[
            pl.BlockSpec((BLK, 128), lambda i: (i, 0)),
            pl.BlockSpec((128, 128), lambda i: (0, 0)),
            pl.BlockSpec((2, BLK, 16), lambda i: (0, i, 0)),
        ],
        out_specs=[
            pl.BlockSpec((BLK, 128), lambda i: (i, 0)),
            pl.BlockSpec((BLK, 16), lambda i: (i, 0)),
        ],
        out_shape=[
            jax.ShapeDtypeStruct((N, 128), jnp.float32),
            jax.ShapeDtypeStruct((N, 16), jnp.float32),
        ],
    )(x, w, degp)


def _tc_mid(p, hp, dis, b, w, dout):
    return pl.pallas_call(
        _tc_mid_body,
        grid=(GRID,),
        in_specs=[
            pl.BlockSpec((2, BLK, 128), lambda i: (0, i, 0)),
            pl.BlockSpec((BLK, 128), lambda i: (i, 0)),
            pl.BlockSpec((BLK, 16), lambda i: (i, 0)),
            pl.BlockSpec((1, 128), lambda i: (0, 0)),
            pl.BlockSpec((128, dout), lambda i: (0, 0)),
        ],
        out_specs=pl.BlockSpec((BLK, dout), lambda i: (i, 0)),
        out_shape=jax.ShapeDtypeStruct((N, dout), jnp.float32),
    )(p, hp, dis, b, w)


def _tc_final(p, hp, dis, b):
    return pl.pallas_call(
        _tc_final_body,
        grid=(GRID,),
        in_specs=[
            pl.BlockSpec((2, BLK, 64), lambda i: (0, i, 0)),
            pl.BlockSpec((BLK, 64), lambda i: (i, 0)),
            pl.BlockSpec((BLK, 16), lambda i: (i, 0)),
            pl.BlockSpec((1, 64), lambda i: (0, 0)),
        ],
        out_specs=pl.BlockSpec((BLK, 64), lambda i: (i, 0)),
        out_shape=jax.ShapeDtypeStruct((N, 64), jnp.float32),
    )(p, hp, dis, b)


def kernel(x, edge_index, W1, b1, W2, b2, W3, b3, W4, b4, W5, b5):
    row2d = edge_index[0].reshape(NCH, C)
    col2d = edge_index[1].reshape(NCH, C)
    z128 = jnp.zeros((SZ, 128), jnp.float32)
    z64 = jnp.zeros((SZ, 64), jnp.float32)
    z16 = jnp.zeros((SZ, 16), jnp.float32)
    ones16 = jnp.ones((C, 16), jnp.float32)

    degp = _make_deg()(col2d, z16, ones16)
    hp1, dis = _tc1(x, W1, degp)
    agg = _make_agg128()
    p = agg(hp1, row2d, col2d, z128)
    hp2 = _tc_mid(p, hp1, dis, b1.reshape(1, 128), W2, 128)
    p = agg(hp2, row2d, col2d, z128)
    hp3 = _tc_mid(p, hp2, dis, b2.reshape(1, 128), W3, 128)
    p = agg(hp3, row2d, col2d, z128)
    hp4 = _tc_mid(p, hp3, dis, b3.reshape(1, 128), W4, 128)
    p = agg(hp4, row2d, col2d, z128)
    hp5 = _tc_mid(p, hp4, dis, b4.reshape(1, 128), W5, 64)
    p = _make_agg64()(hp5, row2d, col2d, z64)
    return _tc_final(p, hp5, dis, b5.reshape(1, 64))


# speedup vs baseline: 1.0068x; 1.0068x over previous
import functools

import jax
import jax.numpy as jnp
from jax import lax
from jax.experimental import pallas as pl
from jax.experimental.pallas import tpu as pltpu
from jax.experimental.pallas import tpu_sc as plsc

N = 10000
E = 320000
NC = 2
NS = 16
NW = NC * NS
C = 125
NCH = E // C
CH_E = NCH // NW
G = CH_E // 2
SZ = 632
NP = NS * SZ
BLK = 1000
GRID = N // BLK

_MESH = dict(core_axis_name="c", subcore_axis_name="s")


def _deg_body(col_hbm, zeros_hbm, ones_hbm, out_hbm, colbuf, ones_v, acc, sem0, sem1):
    c = lax.axis_index("c")
    s = lax.axis_index("s")
    w = s * NC + c
    pltpu.async_copy(zeros_hbm, acc.at[pl.ds(s * SZ, SZ)], sem0)
    pltpu.sync_copy(col_hbm.at[pl.ds(w * CH_E, CH_E)], colbuf)
    pltpu.sync_copy(ones_hbm, ones_v)
    pltpu.make_async_copy(zeros_hbm, acc.at[pl.ds(s * SZ, SZ)], sem0).wait()
    plsc.subcore_barrier()

    pltpu.async_copy(ones_v, acc.at[colbuf.at[0]], sem0, add=True)

    def body(k, carry):
        i = 2 * k
        pltpu.async_copy(ones_v, acc.at[colbuf.at[i + 1]], sem1, add=True)
        pltpu.make_async_copy(ones_v, acc.at[colbuf.at[i]], sem0).wait()

        @pl.when(k + 1 < CH_E // 2)
        def _():
            pltpu.async_copy(ones_v, acc.at[colbuf.at[i + 2]], sem0, add=True)

        pltpu.make_async_copy(ones_v, acc.at[colbuf.at[i + 1]], sem1).wait()
        return carry

    lax.fori_loop(0, CH_E // 2, body, 0)
    plsc.subcore_barrier()
    pltpu.sync_copy(acc.at[pl.ds(s * SZ, SZ)], out_hbm.at[c, pl.ds(s * SZ, SZ)])


def _make_deg():
    return pl.kernel(
        _deg_body,
        out_type=jax.ShapeDtypeStruct((NC, NP, 16), jnp.float32),
        mesh=plsc.VectorSubcoreMesh(**_MESH),
        compiler_params=pltpu.CompilerParams(use_tc_tiling_on_sc=False),
        scratch_types=[
            pltpu.VMEM((CH_E, C), jnp.int32),
            pltpu.VMEM((C, 16), jnp.float32),
            pltpu.VMEM_SHARED((NP, 16), jnp.float32),
            pltpu.SemaphoreType.DMA,
            pltpu.SemaphoreType.DMA,
        ],
    )


def _agg_loop(table, rowbuf, colbuf, rows0, rows1, acc, sem0, sem1, nchunks):
    pltpu.async_copy(table.at[rowbuf.at[0]], rows0, sem0)

    def body(k, carry):
        i = 2 * k
        pltpu.async_copy(table.at[rowbuf.at[i + 1]], rows1, sem1)
        pltpu.make_async_copy(table.at[rowbuf.at[i]], rows0, sem0).wait()
        pltpu.sync_copy(rows0, acc.at[colbuf.at[i]], add=True)

        @pl.when(k + 1 < nchunks // 2)
        def _():
            pltpu.async_copy(table.at[rowbuf.at[i + 2]], rows0, sem0)

        pltpu.make_async_copy(table.at[rowbuf.at[i + 1]], rows1, sem1).wait()
        pltpu.sync_copy(rows1, acc.at[colbuf.at[i + 1]], add=True)
        return carry

    lax.fori_loop(0, nchunks // 2, body, 0)


def _agg128_body(hp_hbm, row_hbm, col_hbm, zeros_hbm, out_hbm,
                 rowbuf, colbuf, rows0, rows1, acc, sem0, sem1):
    c = lax.axis_index("c")
    s = lax.axis_index("s")
    w = s * NC + c
    pltpu.async_copy(zeros_hbm, acc.at[pl.ds(s * SZ, SZ)], sem0)
    pltpu.sync_copy(row_hbm.at[pl.ds(w * CH_E, G)], rowbuf)
    pltpu.sync_copy(col_hbm.at[pl.ds(w * CH_E, G)], colbuf)
    pltpu.make_async_copy(zeros_hbm, acc.at[pl.ds(s * SZ, SZ)], sem0).wait()
    plsc.subcore_barrier()
    for g in range(2):
        if g:
            base = w * CH_E + g * G
            pltpu.sync_copy(row_hbm.at[pl.ds(base, G)], rowbuf)
            pltpu.sync_copy(col_hbm.at[pl.ds(base, G)], colbuf)
        _agg_loop(hp_hbm, rowbuf, colbuf, rows0, rows1, acc, sem0, sem1, G)
    plsc.subcore_barrier()
    pltpu.sync_copy(acc.at[pl.ds(s * SZ, SZ)], out_hbm.at[c, pl.ds(s * SZ, SZ)])


def _make_agg128():
    return pl.kernel(
        _agg128_body,
        out_type=jax.ShapeDtypeStruct((NC, NP, 128), jnp.float32),
        mesh=plsc.VectorSubcoreMesh(**_MESH),
        scratch_types=[
            pltpu.VMEM((G, C), jnp.int32),
            pltpu.VMEM((G, C), jnp.int32),
            pltpu.VMEM((C, 128), jnp.float32),
            pltpu.VMEM((C, 128), jnp.float32),
            pltpu.VMEM_SHARED((NP, 128), jnp.float32),
            pltpu.SemaphoreType.DMA,
            pltpu.SemaphoreType.DMA,
        ],
    )


def _agg64_body(hp_hbm, row_hbm, col_hbm, zeros_hbm, out_hbm,
                rowbuf, colbuf, rows0, rows1, acc, sem0, sem1):
    c = lax.axis_index("c")
    s = lax.axis_index("s")
    w = s * NC + c
    pltpu.async_copy(zeros_hbm, acc.at[pl.ds(s * SZ, SZ)], sem0)
    pltpu.sync_copy(row_hbm.at[pl.ds(w * CH_E, CH_E)], rowbuf)
    pltpu.sync_copy(col_hbm.at[pl.ds(w * CH_E, CH_E)], colbuf)
    pltpu.make_async_copy(zeros_hbm, acc.at[pl.ds(s * SZ, SZ)], sem0).wait()
    plsc.subcore_barrier()
    _agg_loop(hp_hbm, rowbuf, colbuf, rows0, rows1, acc, sem0, sem1, CH_E)
    plsc.subcore_barrier()
    pltpu.sync_copy(acc.at[pl.ds(s * SZ, SZ)], out_hbm.at[c, pl.ds(s * SZ, SZ)])


def _make_agg64():
    return pl.kernel(
        _agg64_body,
        out_type=jax.ShapeDtypeStruct((NC, NP, 64), jnp.float32),
        mesh=plsc.VectorSubcoreMesh(**_MESH),
        compiler_params=pltpu.CompilerParams(use_tc_tiling_on_sc=False),
        scratch_types=[
            pltpu.VMEM((CH_E, C), jnp.int32),
            pltpu.VMEM((CH_E, C), jnp.int32),
            pltpu.VMEM((C, 64), jnp.float32),
            pltpu.VMEM((C, 64), jnp.float32),
            pltpu.VMEM_SHARED((NP, 64), jnp.float32),
            pltpu.SemaphoreType.DMA,
            pltpu.SemaphoreType.DMA,
        ],
    )


def _tc1_body(x_ref, w_ref, degp_ref, hp_ref, dis_ref):
    d = degp_ref[...]
    deg = 1.0 + d[0] + d[1]
    dis16 = lax.rsqrt(deg)
    h = jnp.dot(x_ref[...], w_ref[...], preferred_element_type=jnp.float32)
    hp_ref[...] = h * dis16[:, 0:1]
    dis_ref[...] = dis16


def _tc_mid_body(p_ref, hp_ref, dis_ref, b_ref, w_ref, o_ref):
    p = p_ref[...]
    dis = dis_ref[...][:, 0:1]
    z = (p[0] + p[1] + hp_ref[...]) * dis + b_ref[...]
    a = jnp.maximum(z, 0.0)
    o_ref[...] = jnp.dot(a, w_ref[...], preferred_element_type=jnp.float32) * dis


def _tc_final_body(p_ref, hp_ref, dis_ref, b_ref, o_ref):
    p = p_ref[...]
    dis = dis_ref[...][:, 0:1]
    o_ref[...] = (p[0] + p[1] + hp_ref[...]) * dis + b_ref[...]


def _tc1(x, w, degp):
    return pl.pallas_call(
        _tc1_body,
        grid=(GRID,),
        in_specs=[
            pl.BlockSpec((BLK, 128), lambda i: (i, 0)),
            pl.BlockSpec((128, 128), lambda i: (0, 0)),
            pl.BlockSpec((2, BLK, 16), lambda i: (0, i, 0)),
        ],
        out_specs=[
            pl.BlockSpec((BLK, 128), lambda i: (i, 0)),
            pl.BlockSpec((BLK, 16), lambda i: (i, 0)),
        ],
        out_shape=[
            jax.ShapeDtypeStruct((N, 128), jnp.float32),
            jax.ShapeDtypeStruct((N, 16), jnp.float32),
        ],
    )(x, w, degp)


def _tc_mid(p, hp, dis, b, w, dout):
    return pl.pallas_call(
        _tc_mid_body,
        grid=(GRID,),
        in_specs=[
            pl.BlockSpec((2, BLK, 128), lambda i: (0, i, 0)),
            pl.BlockSpec((BLK, 128), lambda i: (i, 0)),
            pl.BlockSpec((BLK, 16), lambda i: (i, 0)),
            pl.BlockSpec((1, 128), lambda i: (0, 0)),
            pl.BlockSpec((128, dout), lambda i: (0, 0)),
        ],
        out_specs=pl.BlockSpec((BLK, dout), lambda i: (i, 0)),
        out_shape=jax.ShapeDtypeStruct((N, dout), jnp.float32),
    )(p, hp, dis, b, w)


def _tc_final(p, hp, dis, b):
    return pl.pallas_call(
        _tc_final_body,
        grid=(GRID,),
        in_specs=[
            pl.BlockSpec((2, BLK, 64), lambda i: (0, i, 0)),
            pl.BlockSpec((BLK, 64), lambda i: (i, 0)),
            pl.BlockSpec((BLK, 16), lambda i: (i, 0)),
            pl.BlockSpec((1, 64), lambda i: (0, 0)),
        ],
        out_specs=pl.BlockSpec((BLK, 64), lambda i: (i, 0)),
        out_shape=jax.ShapeDtypeStruct((N, 64), jnp.float32),
    )(p, hp, dis, b)


def kernel(x, edge_index, W1, b1, W2, b2, W3, b3, W4, b4, W5, b5):
    row2d = edge_index[0].reshape(NCH, C)
    col2d = edge_index[1].reshape(NCH, C)
    z128 = jnp.zeros((SZ, 128), jnp.float32)
    z64 = jnp.zeros((SZ, 64), jnp.float32)
    z16 = jnp.zeros((SZ, 16), jnp.float32)
    ones16 = jnp.ones((C, 16), jnp.float32)

    degp = _make_deg()(col2d, z16, ones16)
    hp1, dis = _tc1(x, W1, degp)
    agg = _make_agg128()
    p = agg(hp1, row2d, col2d, z128)
    hp2 = _tc_mid(p, hp1, dis, b1.reshape(1, 128), W2, 128)
    p = agg(hp2, row2d, col2d, z128)
    hp3 = _tc_mid(p, hp2, dis, b2.reshape(1, 128), W3, 128)
    p = agg(hp3, row2d, col2d, z128)
    hp4 = _tc_mid(p, hp3, dis, b3.reshape(1, 128), W4, 128)
    p = agg(hp4, row2d, col2d, z128)
    hp5 = _tc_mid(p, hp4, dis, b4.reshape(1, 128), W5, 64)
    p = _make_agg64()(hp5, row2d, col2d, z64)
    return _tc_final(p, hp5, dis, b5.reshape(1, 64))


# speedup vs baseline: 1.0292x; 1.0222x over previous
import functools

import jax
import jax.numpy as jnp
from jax import lax
from jax.experimental import pallas as pl
from jax.experimental.pallas import tpu as pltpu
from jax.experimental.pallas import tpu_sc as plsc

N = 10000
E = 320000
NC = 2
NS = 16
NW = NC * NS
C = 125
NCH = E // C
CH_E = NCH // NW
G = CH_E // 2
SZ = 632
NP = NS * SZ
BLK = 2000
GRID = N // BLK

_MESH = dict(core_axis_name="c", subcore_axis_name="s")


def _deg_body(col_hbm, zeros_hbm, ones_hbm, out_hbm, colbuf, ones_v, acc, sem0, sem1):
    c = lax.axis_index("c")
    s = lax.axis_index("s")
    w = s * NC + c
    pltpu.async_copy(zeros_hbm, acc.at[pl.ds(s * SZ, SZ)], sem0)
    pltpu.sync_copy(col_hbm.at[pl.ds(w * CH_E, CH_E)], colbuf)
    pltpu.sync_copy(ones_hbm, ones_v)
    pltpu.make_async_copy(zeros_hbm, acc.at[pl.ds(s * SZ, SZ)], sem0).wait()
    plsc.subcore_barrier()

    pltpu.async_copy(ones_v, acc.at[colbuf.at[0]], sem0, add=True)

    def body(k, carry):
        i = 2 * k
        pltpu.async_copy(ones_v, acc.at[colbuf.at[i + 1]], sem1, add=True)
        pltpu.make_async_copy(ones_v, acc.at[colbuf.at[i]], sem0).wait()

        @pl.when(k + 1 < CH_E // 2)
        def _():
            pltpu.async_copy(ones_v, acc.at[colbuf.at[i + 2]], sem0, add=True)

        pltpu.make_async_copy(ones_v, acc.at[colbuf.at[i + 1]], sem1).wait()
        return carry

    lax.fori_loop(0, CH_E // 2, body, 0)
    plsc.subcore_barrier()
    pltpu.sync_copy(acc.at[pl.ds(s * SZ, SZ)], out_hbm.at[c, pl.ds(s * SZ, SZ)])


def _make_deg():
    return pl.kernel(
        _deg_body,
        out_type=jax.ShapeDtypeStruct((NC, NP, 16), jnp.float32),
        mesh=plsc.VectorSubcoreMesh(**_MESH),
        compiler_params=pltpu.CompilerParams(use_tc_tiling_on_sc=False),
        scratch_types=[
            pltpu.VMEM((CH_E, C), jnp.int32),
            pltpu.VMEM((C, 16), jnp.float32),
            pltpu.VMEM_SHARED((NP, 16), jnp.float32),
            pltpu.SemaphoreType.DMA,
            pltpu.SemaphoreType.DMA,
        ],
    )


def _agg_loop(table, rowbuf, colbuf, rows0, rows1, acc, sem0, sem1, nchunks):
    pltpu.async_copy(table.at[rowbuf.at[0]], rows0, sem0)

    def body(k, carry):
        i = 2 * k
        pltpu.async_copy(table.at[rowbuf.at[i + 1]], rows1, sem1)
        pltpu.make_async_copy(table.at[rowbuf.at[i]], rows0, sem0).wait()
        pltpu.sync_copy(rows0, acc.at[colbuf.at[i]], add=True)

        @pl.when(k + 1 < nchunks // 2)
        def _():
            pltpu.async_copy(table.at[rowbuf.at[i + 2]], rows0, sem0)

        pltpu.make_async_copy(table.at[rowbuf.at[i + 1]], rows1, sem1).wait()
        pltpu.sync_copy(rows1, acc.at[colbuf.at[i + 1]], add=True)
        return carry

    lax.fori_loop(0, nchunks // 2, body, 0)


def _agg128_body(hp_hbm, row_hbm, col_hbm, zeros_hbm, out_hbm,
                 rowbuf, colbuf, rows0, rows1, acc, sem0, sem1):
    c = lax.axis_index("c")
    s = lax.axis_index("s")
    w = s * NC + c
    pltpu.async_copy(zeros_hbm, acc.at[pl.ds(s * SZ, SZ)], sem0)
    pltpu.sync_copy(row_hbm.at[pl.ds(w * CH_E, G)], rowbuf)
    pltpu.sync_copy(col_hbm.at[pl.ds(w * CH_E, G)], colbuf)
    pltpu.make_async_copy(zeros_hbm, acc.at[pl.ds(s * SZ, SZ)], sem0).wait()
    plsc.subcore_barrier()
    for g in range(2):
        if g:
            base = w * CH_E + g * G
            pltpu.sync_copy(row_hbm.at[pl.ds(base, G)], rowbuf)
            pltpu.sync_copy(col_hbm.at[pl.ds(base, G)], colbuf)
        _agg_loop(hp_hbm, rowbuf, colbuf, rows0, rows1, acc, sem0, sem1, G)
    plsc.subcore_barrier()
    pltpu.sync_copy(acc.at[pl.ds(s * SZ, SZ)], out_hbm.at[c, pl.ds(s * SZ, SZ)])


def _make_agg128():
    return pl.kernel(
        _agg128_body,
        out_type=jax.ShapeDtypeStruct((NC, NP, 128), jnp.float32),
        mesh=plsc.VectorSubcoreMesh(**_MESH),
        scratch_types=[
            pltpu.VMEM((G, C), jnp.int32),
            pltpu.VMEM((G, C), jnp.int32),
            pltpu.VMEM((C, 128), jnp.float32),
            pltpu.VMEM((C, 128), jnp.float32),
            pltpu.VMEM_SHARED((NP, 128), jnp.float32),
            pltpu.SemaphoreType.DMA,
            pltpu.SemaphoreType.DMA,
        ],
    )


def _agg64_body(hp_hbm, row_hbm, col_hbm, zeros_hbm, out_hbm,
                rowbuf, colbuf, rows0, rows1, acc, sem0, sem1):
    c = lax.axis_index("c")
    s = lax.axis_index("s")
    w = s * NC + c
    pltpu.async_copy(zeros_hbm, acc.at[pl.ds(s * SZ, SZ)], sem0)
    pltpu.sync_copy(row_hbm.at[pl.ds(w * CH_E, CH_E)], rowbuf)
    pltpu.sync_copy(col_hbm.at[pl.ds(w * CH_E, CH_E)], colbuf)
    pltpu.make_async_copy(zeros_hbm, acc.at[pl.ds(s * SZ, SZ)], sem0).wait()
    plsc.subcore_barrier()
    _agg_loop(hp_hbm, rowbuf, colbuf, rows0, rows1, acc, sem0, sem1, CH_E)
    plsc.subcore_barrier()
    pltpu.sync_copy(acc.at[pl.ds(s * SZ, SZ)], out_hbm.at[c, pl.ds(s * SZ, SZ)])


def _make_agg64():
    return pl.kernel(
        _agg64_body,
        out_type=jax.ShapeDtypeStruct((NC, NP, 64), jnp.float32),
        mesh=plsc.VectorSubcoreMesh(**_MESH),
        compiler_params=pltpu.CompilerParams(use_tc_tiling_on_sc=False),
        scratch_types=[
            pltpu.VMEM((CH_E, C), jnp.int32),
            pltpu.VMEM((CH_E, C), jnp.int32),
            pltpu.VMEM((C, 64), jnp.float32),
            pltpu.VMEM((C, 64), jnp.float32),
            pltpu.VMEM_SHARED((NP, 64), jnp.float32),
            pltpu.SemaphoreType.DMA,
            pltpu.SemaphoreType.DMA,
        ],
    )


def _tc1_body(x_ref, w_ref, degp_ref, hp_ref, dis_ref):
    d = degp_ref[...]
    deg = 1.0 + d[0] + d[1]
    dis16 = lax.rsqrt(deg)
    h = jnp.dot(x_ref[...], w_ref[...], preferred_element_type=jnp.float32)
    hp_ref[...] = h * dis16[:, 0:1]
    dis_ref[...] = dis16


def _tc_mid_body(p_ref, hp_ref, dis_ref, b_ref, w_ref, o_ref):
    p = p_ref[...]
    dis = dis_ref[...][:, 0:1]
    z = (p[0] + p[1] + hp_ref[...]) * dis + b_ref[...]
    a = jnp.maximum(z, 0.0)
    o_ref[...] = jnp.dot(a, w_ref[...], preferred_element_type=jnp.float32) * dis


def _tc_final_body(p_ref, hp_ref, dis_ref, b_ref, o_ref):
    p = p_ref[...]
    dis = dis_ref[...][:, 0:1]
    o_ref[...] = (p[0] + p[1] + hp_ref[...]) * dis + b_ref[...]


def _tc1(x, w, degp):
    return pl.pallas_call(
        _tc1_body,
        grid=(GRID,),
        in_specs=[
            pl.BlockSpec((BLK, 128), lambda i: (i, 0)),
            pl.BlockSpec((128, 128), lambda i: (0, 0)),
            pl.BlockSpec((2, BLK, 16), lambda i: (0, i, 0)),
        ],
        out_specs=[
            pl.BlockSpec((BLK, 128), lambda i: (i, 0)),
            pl.BlockSpec((BLK, 16), lambda i: (i, 0)),
        ],
        out_shape=[
            jax.ShapeDtypeStruct((N, 128), jnp.float32),
            jax.ShapeDtypeStruct((N, 16), jnp.float32),
        ],
    )(x, w, degp)


def _tc_mid(p, hp, dis, b, w, dout):
    return pl.pallas_call(
        _tc_mid_body,
        grid=(GRID,),
        in_specs=[
            pl.BlockSpec((2, BLK, 128), lambda i: (0, i, 0)),
            pl.BlockSpec((BLK, 128), lambda i: (i, 0)),
            pl.BlockSpec((BLK, 16), lambda i: (i, 0)),
            pl.BlockSpec((1, 128), lambda i: (0, 0)),
            pl.BlockSpec((128, dout), lambda i: (0, 0)),
        ],
        out_specs=pl.BlockSpec((BLK, dout), lambda i: (i, 0)),
        out_shape=jax.ShapeDtypeStruct((N, dout), jnp.float32),
    )(p, hp, dis, b, w)


def _tc_final(p, hp, dis, b):
    return pl.pallas_call(
        _tc_final_body,
        grid=(GRID,),
        in_specs=[
            pl.BlockSpec((2, BLK, 64), lambda i: (0, i, 0)),
            pl.BlockSpec((BLK, 64), lambda i: (i, 0)),
            pl.BlockSpec((BLK, 16), lambda i: (i, 0)),
            pl.BlockSpec((1, 64), lambda i: (0, 0)),
        ],
        out_specs=pl.BlockSpec((BLK, 64), lambda i: (i, 0)),
        out_shape=jax.ShapeDtypeStruct((N, 64), jnp.float32),
    )(p, hp, dis, b)


def kernel(x, edge_index, W1, b1, W2, b2, W3, b3, W4, b4, W5, b5):
    row2d = edge_index[0].reshape(NCH, C)
    col2d = edge_index[1].reshape(NCH, C)
    z128 = jnp.zeros((SZ, 128), jnp.float32)
    z64 = jnp.zeros((SZ, 64), jnp.float32)
    z16 = jnp.zeros((SZ, 16), jnp.float32)
    ones16 = jnp.ones((C, 16), jnp.float32)

    degp = _make_deg()(col2d, z16, ones16)
    hp1, dis = _tc1(x, W1, degp)
    agg = _make_agg128()
    p = agg(hp1, row2d, col2d, z128)
    hp2 = _tc_mid(p, hp1, dis, b1.reshape(1, 128), W2, 128)
    p = agg(hp2, row2d, col2d, z128)
    hp3 = _tc_mid(p, hp2, dis, b2.reshape(1, 128), W3, 128)
    p = agg(hp3, row2d, col2d, z128)
    hp4 = _tc_mid(p, hp3, dis, b3.reshape(1, 128), W4, 128)
    p = agg(hp4, row2d, col2d, z128)
    hp5 = _tc_mid(p, hp4, dis, b4.reshape(1, 128), W5, 64)
    p = _make_agg64()(hp5, row2d, col2d, z64)
    return _tc_final(p, hp5, dis, b5.reshape(1, 64))


# speedup vs baseline: 1.0324x; 1.0031x over previous
import functools

import jax
import jax.numpy as jnp
from jax import lax
from jax.experimental import pallas as pl
from jax.experimental.pallas import tpu as pltpu
from jax.experimental.pallas import tpu_sc as plsc

N = 10000
E = 320000
NC = 2
NS = 16
NW = NC * NS
C = 125
NCH = E // C
CH_E = NCH // NW
G = CH_E // 2
SZ = 632
NP = NS * SZ
BLK = 5000
GRID = N // BLK

_MESH = dict(core_axis_name="c", subcore_axis_name="s")


def _deg_body(col_hbm, zeros_hbm, ones_hbm, out_hbm, colbuf, ones_v, acc, sem0, sem1):
    c = lax.axis_index("c")
    s = lax.axis_index("s")
    w = s * NC + c
    pltpu.async_copy(zeros_hbm, acc.at[pl.ds(s * SZ, SZ)], sem0)
    pltpu.sync_copy(col_hbm.at[pl.ds(w * CH_E, CH_E)], colbuf)
    pltpu.sync_copy(ones_hbm, ones_v)
    pltpu.make_async_copy(zeros_hbm, acc.at[pl.ds(s * SZ, SZ)], sem0).wait()
    plsc.subcore_barrier()

    pltpu.async_copy(ones_v, acc.at[colbuf.at[0]], sem0, add=True)

    def body(k, carry):
        i = 2 * k
        pltpu.async_copy(ones_v, acc.at[colbuf.at[i + 1]], sem1, add=True)
        pltpu.make_async_copy(ones_v, acc.at[colbuf.at[i]], sem0).wait()

        @pl.when(k + 1 < CH_E // 2)
        def _():
            pltpu.async_copy(ones_v, acc.at[colbuf.at[i + 2]], sem0, add=True)

        pltpu.make_async_copy(ones_v, acc.at[colbuf.at[i + 1]], sem1).wait()
        return carry

    lax.fori_loop(0, CH_E // 2, body, 0)
    plsc.subcore_barrier()
    pltpu.sync_copy(acc.at[pl.ds(s * SZ, SZ)], out_hbm.at[c, pl.ds(s * SZ, SZ)])


def _make_deg():
    return pl.kernel(
        _deg_body,
        out_type=jax.ShapeDtypeStruct((NC, NP, 16), jnp.float32),
        mesh=plsc.VectorSubcoreMesh(**_MESH),
        compiler_params=pltpu.CompilerParams(use_tc_tiling_on_sc=False),
        scratch_types=[
            pltpu.VMEM((CH_E, C), jnp.int32),
            pltpu.VMEM((C, 16), jnp.float32),
            pltpu.VMEM_SHARED((NP, 16), jnp.float32),
            pltpu.SemaphoreType.DMA,
            pltpu.SemaphoreType.DMA,
        ],
    )


def _agg_loop(table, rowbuf, colbuf, rows0, rows1, acc, sem0, sem1, nchunks):
    pltpu.async_copy(table.at[rowbuf.at[0]], rows0, sem0)

    def body(k, carry):
        i = 2 * k
        pltpu.async_copy(table.at[rowbuf.at[i + 1]], rows1, sem1)
        pltpu.make_async_copy(table.at[rowbuf.at[i]], rows0, sem0).wait()
        pltpu.sync_copy(rows0, acc.at[colbuf.at[i]], add=True)

        @pl.when(k + 1 < nchunks // 2)
        def _():
            pltpu.async_copy(table.at[rowbuf.at[i + 2]], rows0, sem0)

        pltpu.make_async_copy(table.at[rowbuf.at[i + 1]], rows1, sem1).wait()
        pltpu.sync_copy(rows1, acc.at[colbuf.at[i + 1]], add=True)
        return carry

    lax.fori_loop(0, nchunks // 2, body, 0)


def _agg128_body(hp_hbm, row_hbm, col_hbm, zeros_hbm, out_hbm,
                 rowbuf, colbuf, rows0, rows1, acc, sem0, sem1):
    c = lax.axis_index("c")
    s = lax.axis_index("s")
    w = s * NC + c
    pltpu.async_copy(zeros_hbm, acc.at[pl.ds(s * SZ, SZ)], sem0)
    pltpu.sync_copy(row_hbm.at[pl.ds(w * CH_E, G)], rowbuf)
    pltpu.sync_copy(col_hbm.at[pl.ds(w * CH_E, G)], colbuf)
    pltpu.make_async_copy(zeros_hbm, acc.at[pl.ds(s * SZ, SZ)], sem0).wait()
    plsc.subcore_barrier()
    for g in range(2):
        if g:
            base = w * CH_E + g * G
            pltpu.sync_copy(row_hbm.at[pl.ds(base, G)], rowbuf)
            pltpu.sync_copy(col_hbm.at[pl.ds(base, G)], colbuf)
        _agg_loop(hp_hbm, rowbuf, colbuf, rows0, rows1, acc, sem0, sem1, G)
    plsc.subcore_barrier()
    pltpu.sync_copy(acc.at[pl.ds(s * SZ, SZ)], out_hbm.at[c, pl.ds(s * SZ, SZ)])


def _make_agg128():
    return pl.kernel(
        _agg128_body,
        out_type=jax.ShapeDtypeStruct((NC, NP, 128), jnp.float32),
        mesh=plsc.VectorSubcoreMesh(**_MESH),
        scratch_types=[
            pltpu.VMEM((G, C), jnp.int32),
            pltpu.VMEM((G, C), jnp.int32),
            pltpu.VMEM((C, 128), jnp.float32),
            pltpu.VMEM((C, 128), jnp.float32),
            pltpu.VMEM_SHARED((NP, 128), jnp.float32),
            pltpu.SemaphoreType.DMA,
            pltpu.SemaphoreType.DMA,
        ],
    )


def _agg64_body(hp_hbm, row_hbm, col_hbm, zeros_hbm, out_hbm,
                rowbuf, colbuf, rows0, rows1, acc, sem0, sem1):
    c = lax.axis_index("c")
    s = lax.axis_index("s")
    w = s * NC + c
    pltpu.async_copy(zeros_hbm, acc.at[pl.ds(s * SZ, SZ)], sem0)
    pltpu.sync_copy(row_hbm.at[pl.ds(w * CH_E, CH_E)], rowbuf)
    pltpu.sync_copy(col_hbm.at[pl.ds(w * CH_E, CH_E)], colbuf)
    pltpu.make_async_copy(zeros_hbm, acc.at[pl.ds(s * SZ, SZ)], sem0).wait()
    plsc.subcore_barrier()
    _agg_loop(hp_hbm, rowbuf, colbuf, rows0, rows1, acc, sem0, sem1, CH_E)
    plsc.subcore_barrier()
    pltpu.sync_copy(acc.at[pl.ds(s * SZ, SZ)], out_hbm.at[c, pl.ds(s * SZ, SZ)])


def _make_agg64():
    return pl.kernel(
        _agg64_body,
        out_type=jax.ShapeDtypeStruct((NC, NP, 64), jnp.float32),
        mesh=plsc.VectorSubcoreMesh(**_MESH),
        compiler_params=pltpu.CompilerParams(use_tc_tiling_on_sc=False),
        scratch_types=[
            pltpu.VMEM((CH_E, C), jnp.int32),
            pltpu.VMEM((CH_E, C), jnp.int32),
            pltpu.VMEM((C, 64), jnp.float32),
            pltpu.VMEM((C, 64), jnp.float32),
            pltpu.VMEM_SHARED((NP, 64), jnp.float32),
            pltpu.SemaphoreType.DMA,
            pltpu.SemaphoreType.DMA,
        ],
    )


def _tc1_body(x_ref, w_ref, degp_ref, hp_ref, dis_ref):
    d = degp_ref[...]
    deg = 1.0 + d[0] + d[1]
    dis16 = lax.rsqrt(deg)
    h = jnp.dot(x_ref[...], w_ref[...], preferred_element_type=jnp.float32)
    hp_ref[...] = h * dis16[:, 0:1]
    dis_ref[...] = dis16


def _tc_mid_body(p_ref, hp_ref, dis_ref, b_ref, w_ref, o_ref):
    p = p_ref[...]
    dis = dis_ref[...][:, 0:1]
    z = (p[0] + p[1] + hp_ref[...]) * dis + b_ref[...]
    a = jnp.maximum(z, 0.0)
    o_ref[...] = jnp.dot(a, w_ref[...], preferred_element_type=jnp.float32) * dis


def _tc_final_body(p_ref, hp_ref, dis_ref, b_ref, o_ref):
    p = p_ref[...]
    dis = dis_ref[...][:, 0:1]
    o_ref[...] = (p[0] + p[1] + hp_ref[...]) * dis + b_ref[...]


def _tc1(x, w, degp):
    return pl.pallas_call(
        _tc1_body,
        grid=(GRID,),
        in_specs=[
            pl.BlockSpec((BLK, 128), lambda i: (i, 0)),
            pl.BlockSpec((128, 128), lambda i: (0, 0)),
            pl.BlockSpec((2, BLK, 16), lambda i: (0, i, 0)),
        ],
        out_specs=[
            pl.BlockSpec((BLK, 128), lambda i: (i, 0)),
            pl.BlockSpec((BLK, 16), lambda i: (i, 0)),
        ],
        out_shape=[
            jax.ShapeDtypeStruct((N, 128), jnp.float32),
            jax.ShapeDtypeStruct((N, 16), jnp.float32),
        ],
    )(x, w, degp)


def _tc_mid(p, hp, dis, b, w, dout):
    return pl.pallas_call(
        _tc_mid_body,
        grid=(GRID,),
        in_specs=[
            pl.BlockSpec((2, BLK, 128), lambda i: (0, i, 0)),
            pl.BlockSpec((BLK, 128), lambda i: (i, 0)),
            pl.BlockSpec((BLK, 16), lambda i: (i, 0)),
            pl.BlockSpec((1, 128), lambda i: (0, 0)),
            pl.BlockSpec((128, dout), lambda i: (0, 0)),
        ],
        out_specs=pl.BlockSpec((BLK, dout), lambda i: (i, 0)),
        out_shape=jax.ShapeDtypeStruct((N, dout), jnp.float32),
    )(p, hp, dis, b, w)


def _tc_final(p, hp, dis, b):
    return pl.pallas_call(
        _tc_final_body,
        grid=(GRID,),
        in_specs=[
            pl.BlockSpec((2, BLK, 64), lambda i: (0, i, 0)),
            pl.BlockSpec((BLK, 64), lambda i: (i, 0)),
            pl.BlockSpec((BLK, 16), lambda i: (i, 0)),
            pl.BlockSpec((1, 64), lambda i: (0, 0)),
        ],
        out_specs=pl.BlockSpec((BLK, 64), lambda i: (i, 0)),
        out_shape=jax.ShapeDtypeStruct((N, 64), jnp.float32),
    )(p, hp, dis, b)


def kernel(x, edge_index, W1, b1, W2, b2, W3, b3, W4, b4, W5, b5):
    row2d = edge_index[0].reshape(NCH, C)
    col2d = edge_index[1].reshape(NCH, C)
    z128 = jnp.zeros((SZ, 128), jnp.float32)
    z64 = jnp.zeros((SZ, 64), jnp.float32)
    z16 = jnp.zeros((SZ, 16), jnp.float32)
    ones16 = jnp.ones((C, 16), jnp.float32)

    degp = _make_deg()(col2d, z16, ones16)
    hp1, dis = _tc1(x, W1, degp)
    agg = _make_agg128()
    p = agg(hp1, row2d, col2d, z128)
    hp2 = _tc_mid(p, hp1, dis, b1.reshape(1, 128), W2, 128)
    p = agg(hp2, row2d, col2d, z128)
    hp3 = _tc_mid(p, hp2, dis, b2.reshape(1, 128), W3, 128)
    p = agg(hp3, row2d, col2d, z128)
    hp4 = _tc_mid(p, hp3, dis, b3.reshape(1, 128), W4, 128)
    p = agg(hp4, row2d, col2d, z128)
    hp5 = _tc_mid(p, hp4, dis, b4.reshape(1, 128), W5, 64)
    p = _make_agg64()(hp5, row2d, col2d, z64)
    return _tc_final(p, hp5, dis, b5.reshape(1, 64))
